# baseline (device time: 281250 ns/iter reference)
import jax
import jax.numpy as jnp
from jax import lax
from jax.experimental import pallas as pl
from jax.experimental.pallas import tpu as pltpu

N = 4096
K = 2048
B = 4
SH = 512
NC = 256
NPAIR = (N // NC) // 2


def _body(x_hbm, wo_hbm, out_hbm,
          th_v, mi_v, wo_buf, send_x, recv_x, res_buf, recv_y,
          st_sems, wo_sems, sx_sems, rx_sems, out_sems, ys_sems,
          ry_sems, oy_sems):
    my_x = lax.axis_index("x")
    my_y = lax.axis_index("y")
    xp = (1 - my_x, my_y)
    yp = (my_x, 1 - my_y)

    def jd_off(k):
        return 2 * k * NC + my_y * NC

    def jo_off(k):
        return 2 * k * NC + (1 - my_y) * NC

    st_cp = []
    for b in range(B):
        for dst, row in ((mi_v, b * 2 * SH + my_x * SH),
                         (th_v, b * 2 * SH + (1 - my_x) * SH)):
            cp = pltpu.make_async_copy(
                x_hbm.at[pl.ds(row, SH), :], dst.at[b],
                st_sems.at[len(st_cp) % 2])
            st_cp.append(cp)

    st_cp[0].start()
    st_cp[1].start()

    wo_cp = {}
    wo_cp[0] = pltpu.make_async_copy(
        wo_hbm.at[:, pl.ds(jd_off(0), NC)], wo_buf.at[0], wo_sems.at[0])
    wo_cp[0].start()

    barrier = pltpu.get_barrier_semaphore()
    for nbr in (xp, yp):
        pl.semaphore_signal(barrier, inc=1, device_id=nbr,
                            device_id_type=pl.DeviceIdType.MESH)
    pl.semaphore_wait(barrier, 2)

    for i in range(len(st_cp)):
        st_cp[i].wait()
        if i + 2 < len(st_cp):
            st_cp[i + 2].start()

    rdx = {}
    out_cps = {}
    ysend = {}
    outy = {}

    def process_direct(p):
        rdx[p].wait_recv()
        res_buf[p % 4] += recv_x[p % 4]
        ysend[p] = pltpu.make_async_remote_copy(
            src_ref=res_buf.at[p % 4],
            dst_ref=recv_y.at[p % 4],
            send_sem=ys_sems.at[p % 4],
            recv_sem=ry_sems.at[p % 4],
            device_id=yp,
            device_id_type=pl.DeviceIdType.MESH,
        )
        ysend[p].start()
        out_cps[p] = pltpu.make_async_copy(
            res_buf.at[p % 4], out_hbm.at[:, :, pl.ds(jd_off(p), NC)],
            out_sems.at[p % 4])
        out_cps[p].start()

    def process_fwd(q):
        r = pltpu.make_async_remote_copy(
            src_ref=res_buf.at[q % 4],
            dst_ref=recv_y.at[q % 4],
            send_sem=ys_sems.at[q % 4],
            recv_sem=ry_sems.at[q % 4],
            device_id=yp,
            device_id_type=pl.DeviceIdType.MESH,
        )
        r.wait_recv()
        if q >= 4:
            outy[q - 4].wait()
        outy[q] = pltpu.make_async_copy(
            recv_y.at[q % 4], out_hbm.at[:, :, pl.ds(jo_off(q), NC)],
            oy_sems.at[q % 4])
        outy[q].start()

    for k in range(NPAIR):
        wo_cp[k].wait()
        if k + 1 < NPAIR:
            wo_cp[k + 1] = pltpu.make_async_copy(
                wo_hbm.at[:, pl.ds(jd_off(k + 1), NC)],
                wo_buf.at[(k + 1) % 2], wo_sems.at[(k + 1) % 2])
            wo_cp[k + 1].start()

        if k >= 2:
            rdx[k - 2].wait_send()
        for b in range(B):
            send_x[k % 2, b] = jnp.dot(
                th_v[b], wo_buf[k % 2], preferred_element_type=jnp.float32)
        rdx[k] = pltpu.make_async_remote_copy(
            src_ref=send_x.at[k % 2],
            dst_ref=recv_x.at[k % 4],
            send_sem=sx_sems.at[k % 2],
            recv_sem=rx_sems.at[k % 4],
            device_id=xp,
            device_id_type=pl.DeviceIdType.MESH,
        )
        rdx[k].start()

        if k >= 4:
            out_cps[k - 4].wait()
            ysend[k - 4].wait_send()
        for b in range(B):
            res_buf[k % 4, b] = jnp.dot(
                mi_v[b], wo_buf[k % 2], preferred_element_type=jnp.float32)

        if k >= 1:
            process_direct(k - 1)
        if k >= 2:
            process_fwd(k - 2)

    process_direct(NPAIR - 1)
    process_fwd(NPAIR - 2)
    process_fwd(NPAIR - 1)
    rdx[NPAIR - 2].wait_send()
    rdx[NPAIR - 1].wait_send()
    for p in range(NPAIR - 4, NPAIR):
        out_cps[p].wait()
        ysend[p].wait_send()
        outy[p].wait()


def kernel(O, Wo):
    Bv, S2, H, D = O.shape
    X = O.astype(jnp.bfloat16).reshape(Bv * S2, H * D)
    Wo = Wo.astype(jnp.bfloat16)

    return pl.pallas_call(
        _body,
        out_shape=jax.ShapeDtypeStruct((B, SH, N), jnp.float32),
        in_specs=[
            pl.BlockSpec(memory_space=pl.ANY),
            pl.BlockSpec(memory_space=pl.ANY),
        ],
        out_specs=pl.BlockSpec(memory_space=pl.ANY),
        scratch_shapes=[
            pltpu.VMEM((B, SH, K), jnp.bfloat16),
            pltpu.VMEM((B, SH, K), jnp.bfloat16),
            pltpu.VMEM((2, K, NC), jnp.bfloat16),
            pltpu.VMEM((2, B, SH, NC), jnp.float32),
            pltpu.VMEM((4, B, SH, NC), jnp.float32),
            pltpu.VMEM((4, B, SH, NC), jnp.float32),
            pltpu.VMEM((4, B, SH, NC), jnp.float32),
            pltpu.SemaphoreType.DMA((2,)),
            pltpu.SemaphoreType.DMA((2,)),
            pltpu.SemaphoreType.DMA((2,)),
            pltpu.SemaphoreType.DMA((4,)),
            pltpu.SemaphoreType.DMA((4,)),
            pltpu.SemaphoreType.DMA((4,)),
            pltpu.SemaphoreType.DMA((4,)),
            pltpu.SemaphoreType.DMA((4,)),
        ],
        compiler_params=pltpu.CompilerParams(
            collective_id=0,
            vmem_limit_bytes=62 * 1024 * 1024,
        ),
    )(X, Wo)


# device time: 263446 ns/iter; 1.0676x vs baseline; 1.0676x over previous
import jax
import jax.numpy as jnp
from jax import lax
from jax.experimental import pallas as pl
from jax.experimental.pallas import tpu as pltpu

N = 4096
K = 2048
B = 4
SH = 512
NC = 256
NPAIR = (N // NC) // 2


def _body(x_hbm, wo_hbm, out_hbm,
          th_v, mi_v, wo_buf, send_x, recv_x, res_buf, recv_y,
          st_sems, wo_sems, sx_sems, rx_sems, out_sems, ys_sems,
          ry_sems, oy_sems):
    my_x = lax.axis_index("x")
    my_y = lax.axis_index("y")
    xp = (1 - my_x, my_y)
    yp = (my_x, 1 - my_y)

    def jd_off(k):
        return 2 * k * NC + my_y * NC

    def jo_off(k):
        return 2 * k * NC + (1 - my_y) * NC

    st_cp = []
    for b in range(B):
        for dst, row in ((mi_v, b * 2 * SH + my_x * SH),
                         (th_v, b * 2 * SH + (1 - my_x) * SH)):
            cp = pltpu.make_async_copy(
                x_hbm.at[pl.ds(row, SH), :], dst.at[b],
                st_sems.at[len(st_cp) % 2])
            st_cp.append(cp)

    st_cp[0].start()
    st_cp[1].start()

    wo_cp = {}
    wo_cp[0] = pltpu.make_async_copy(
        wo_hbm.at[:, pl.ds(jd_off(0), NC)], wo_buf.at[0], wo_sems.at[0])
    wo_cp[0].start()

    barrier = pltpu.get_barrier_semaphore()
    for nbr in (xp, yp):
        pl.semaphore_signal(barrier, inc=1, device_id=nbr,
                            device_id_type=pl.DeviceIdType.MESH)
    pl.semaphore_wait(barrier, 2)

    for i in range(len(st_cp)):
        st_cp[i].wait()
        if i + 2 < len(st_cp):
            st_cp[i + 2].start()

    rdx = {}
    out_cps = {}
    ysend = {}
    outy = {}

    def process_direct(p):
        rdx[p].wait_recv()
        res_buf[p % 4] += recv_x[p % 4]
        ysend[p] = pltpu.make_async_remote_copy(
            src_ref=res_buf.at[p % 4],
            dst_ref=recv_y.at[p % 4],
            send_sem=ys_sems.at[p % 4],
            recv_sem=ry_sems.at[p % 4],
            device_id=yp,
            device_id_type=pl.DeviceIdType.MESH,
        )
        ysend[p].start()
        out_cps[p] = pltpu.make_async_copy(
            res_buf.at[p % 4], out_hbm.at[:, :, pl.ds(jd_off(p), NC)],
            out_sems.at[p % 4])
        out_cps[p].start()

    def process_fwd(q):
        r = pltpu.make_async_remote_copy(
            src_ref=res_buf.at[q % 4],
            dst_ref=recv_y.at[q % 4],
            send_sem=ys_sems.at[q % 4],
            recv_sem=ry_sems.at[q % 4],
            device_id=yp,
            device_id_type=pl.DeviceIdType.MESH,
        )
        r.wait_recv()
        if q >= 4:
            outy[q - 4].wait()
        outy[q] = pltpu.make_async_copy(
            recv_y.at[q % 4], out_hbm.at[:, :, pl.ds(jo_off(q), NC)],
            oy_sems.at[q % 4])
        outy[q].start()

    for k in range(NPAIR):
        wo_cp[k].wait()
        if k + 1 < NPAIR:
            wo_cp[k + 1] = pltpu.make_async_copy(
                wo_hbm.at[:, pl.ds(jd_off(k + 1), NC)],
                wo_buf.at[(k + 1) % 2], wo_sems.at[(k + 1) % 2])
            wo_cp[k + 1].start()

        if k >= 2:
            rdx[k - 2].wait_send()
        for b in range(B):
            send_x[k % 2, b] = jnp.dot(
                th_v[b], wo_buf[k % 2], preferred_element_type=jnp.float32)
        rdx[k] = pltpu.make_async_remote_copy(
            src_ref=send_x.at[k % 2],
            dst_ref=recv_x.at[k % 4],
            send_sem=sx_sems.at[k % 2],
            recv_sem=rx_sems.at[k % 4],
            device_id=xp,
            device_id_type=pl.DeviceIdType.MESH,
        )
        rdx[k].start()

        if k >= 4:
            out_cps[k - 4].wait()
            ysend[k - 4].wait_send()
        for b in range(B):
            res_buf[k % 4, b] = jnp.dot(
                mi_v[b], wo_buf[k % 2], preferred_element_type=jnp.float32)

        if k >= 1:
            process_direct(k - 1)
        if k >= 2:
            process_fwd(k - 2)

    process_direct(NPAIR - 1)
    process_fwd(NPAIR - 2)
    process_fwd(NPAIR - 1)
    rdx[NPAIR - 2].wait_send()
    rdx[NPAIR - 1].wait_send()
    for p in range(NPAIR - 4, NPAIR):
        out_cps[p].wait()
        ysend[p].wait_send()
        outy[p].wait()


def kernel(O, Wo):
    Bv, S2, H, D = O.shape
    X = O.astype(jnp.bfloat16).reshape(Bv * S2, H * D)

    return pl.pallas_call(
        _body,
        out_shape=jax.ShapeDtypeStruct((B, SH, N), jnp.float32),
        in_specs=[
            pl.BlockSpec(memory_space=pl.ANY),
            pl.BlockSpec(memory_space=pl.ANY),
        ],
        out_specs=pl.BlockSpec(memory_space=pl.ANY),
        scratch_shapes=[
            pltpu.VMEM((B, SH, K), jnp.bfloat16),
            pltpu.VMEM((B, SH, K), jnp.bfloat16),
            pltpu.VMEM((2, K, NC), jnp.float32),
            pltpu.VMEM((2, B, SH, NC), jnp.float32),
            pltpu.VMEM((4, B, SH, NC), jnp.float32),
            pltpu.VMEM((4, B, SH, NC), jnp.float32),
            pltpu.VMEM((4, B, SH, NC), jnp.float32),
            pltpu.SemaphoreType.DMA((2,)),
            pltpu.SemaphoreType.DMA((2,)),
            pltpu.SemaphoreType.DMA((2,)),
            pltpu.SemaphoreType.DMA((4,)),
            pltpu.SemaphoreType.DMA((4,)),
            pltpu.SemaphoreType.DMA((4,)),
            pltpu.SemaphoreType.DMA((4,)),
            pltpu.SemaphoreType.DMA((4,)),
        ],
        compiler_params=pltpu.CompilerParams(
            collective_id=0,
            vmem_limit_bytes=62 * 1024 * 1024,
        ),
    )(X, Wo)
